# baseline (speedup 1.0000x reference)
import functools

import jax
import jax.numpy as jnp
from jax import lax
from jax.experimental import pallas as pl
from jax.experimental.pallas import tpu as pltpu
from jax.experimental.pallas import tpu_sc as plsc

B, S = 4096, 200
D = 64
NC, NS = 2, 16
NW = NC * NS
B_PER_W = B // NW
NB = 4
GROUPS = B_PER_W // NB
S0 = 128
S1 = S - S0

_mesh = plsc.VectorSubcoreMesh(core_axis_name="c", subcore_axis_name="s")


@functools.partial(
    pl.kernel,
    out_type=jax.ShapeDtypeStruct((B, S, D), jnp.float32),
    mesh=_mesh,
    scratch_types=[
        pltpu.VMEM((B_PER_W, S), jnp.int32),
        pltpu.VMEM((NB, S, D), jnp.float32),
        pltpu.SemaphoreType.DMA((NB,)),
        pltpu.SemaphoreType.DMA((NB,)),
    ],
    compiler_params=pltpu.CompilerParams(use_tc_tiling_on_sc=False),
)
def _gather_kernel(ids_hbm, table_hbm, out_hbm, idx_v, rows_v, gsem, ssem):
    wid = lax.axis_index("s") * NC + lax.axis_index("c")
    b_base = wid * B_PER_W

    pltpu.sync_copy(ids_hbm.at[pl.ds(b_base, B_PER_W)], idx_v)

    def gathers(i, buf):
        pltpu.async_copy(
            table_hbm.at[idx_v.at[i, pl.ds(0, S0)]],
            rows_v.at[buf, pl.ds(0, S0)],
            gsem.at[buf],
        )
        pltpu.async_copy(
            table_hbm.at[idx_v.at[i, pl.ds(S0, S1)]],
            rows_v.at[buf, pl.ds(S0, S1)],
            gsem.at[buf],
        )

    def wait_gathers(i, buf):
        pltpu.make_async_copy(
            table_hbm.at[idx_v.at[i, pl.ds(0, S0)]],
            rows_v.at[buf, pl.ds(0, S0)],
            gsem.at[buf],
        ).wait()
        pltpu.make_async_copy(
            table_hbm.at[idx_v.at[i, pl.ds(S0, S1)]],
            rows_v.at[buf, pl.ds(S0, S1)],
            gsem.at[buf],
        ).wait()

    def store(i, buf):
        return pltpu.async_copy(
            rows_v.at[buf], out_hbm.at[b_base + i], ssem.at[buf]
        )

    def wait_store(i, buf):
        pltpu.make_async_copy(
            rows_v.at[buf], out_hbm.at[b_base + i], ssem.at[buf]
        ).wait()

    for b in range(NB):
        gathers(b, b)

    def group(g, _):
        base = g * NB
        for b in range(NB):
            i = base + b
            wait_gathers(i, b)
            store(i, b)
        for b in range(NB):
            i = base + b
            wait_store(i, b)

            @pl.when(i + NB < B_PER_W)
            def _():
                gathers(i + NB, b)

        return 0

    lax.fori_loop(0, GROUPS, group, 0)


def kernel(zone_ids, table):
    return _gather_kernel(zone_ids.astype(jnp.int32), table)


# speedup vs baseline: 1.2196x; 1.2196x over previous
import functools

import jax
import jax.numpy as jnp
from jax import lax
from jax.experimental import pallas as pl
from jax.experimental.pallas import tpu as pltpu
from jax.experimental.pallas import tpu_sc as plsc

B, S = 4096, 200
D = 64
DP = 128
NC, NS = 2, 16
NW = NC * NS
B_PER_W = B // NW
NB = 3
S0 = 128
S1 = S - S0

_mesh = plsc.VectorSubcoreMesh(core_axis_name="c", subcore_axis_name="s")


@functools.partial(
    pl.kernel,
    out_type=jax.ShapeDtypeStruct((B, S, DP), jnp.float32),
    mesh=_mesh,
    scratch_types=[
        pltpu.VMEM((B_PER_W, S), jnp.int32),
        pltpu.VMEM((NB, S, DP), jnp.float32),
        pltpu.SemaphoreType.DMA((NB,)),
        pltpu.SemaphoreType.DMA((NB,)),
    ],
    compiler_params=pltpu.CompilerParams(use_tc_tiling_on_sc=True),
)
def _gather_kernel(ids_hbm, table_hbm, out_hbm, idx_v, rows_v, gsem, ssem):
    wid = lax.axis_index("s") * NC + lax.axis_index("c")
    b_base = wid * B_PER_W

    pltpu.sync_copy(ids_hbm.at[pl.ds(b_base, B_PER_W)], idx_v)

    def gathers(i, buf):
        pltpu.async_copy(
            table_hbm.at[idx_v.at[i, pl.ds(0, S0)]],
            rows_v.at[buf, pl.ds(0, S0)],
            gsem.at[buf],
        )
        pltpu.async_copy(
            table_hbm.at[idx_v.at[i, pl.ds(S0, S1)]],
            rows_v.at[buf, pl.ds(S0, S1)],
            gsem.at[buf],
        )

    def wait_gathers(i, buf):
        pltpu.make_async_copy(
            table_hbm.at[idx_v.at[i, pl.ds(0, S0)]],
            rows_v.at[buf, pl.ds(0, S0)],
            gsem.at[buf],
        ).wait()
        pltpu.make_async_copy(
            table_hbm.at[idx_v.at[i, pl.ds(S0, S1)]],
            rows_v.at[buf, pl.ds(S0, S1)],
            gsem.at[buf],
        ).wait()

    def store(i, buf):
        return pltpu.async_copy(
            rows_v.at[buf], out_hbm.at[b_base + i], ssem.at[buf]
        )

    def wait_store(i, buf):
        pltpu.make_async_copy(
            rows_v.at[buf], out_hbm.at[b_base + i], ssem.at[buf]
        ).wait()

    for b in range(NB):
        gathers(b, b)

    n_groups = B_PER_W // NB

    def group(g, _):
        base = g * NB
        for b in range(NB):
            i = base + b
            wait_gathers(i, b)
            store(i, b)
        for b in range(NB):
            i = base + b
            wait_store(i, b)

            @pl.when(i + NB < B_PER_W)
            def _():
                gathers(i + NB, b)

        return 0

    lax.fori_loop(0, n_groups, group, 0)

    for i in range(n_groups * NB, B_PER_W):
        b = i % NB
        wait_gathers(i, b)
        store(i, b).wait()


def kernel(zone_ids, table):
    table_p = jnp.pad(table, ((0, 0), (0, DP - D)))
    out_p = _gather_kernel(zone_ids.astype(jnp.int32), table_p)
    return out_p[:, :, :D]
